# Initial kernel scaffold; baseline (speedup 1.0000x reference)
import jax
import jax.numpy as jnp
from jax.experimental import pallas as pl


def kernel(x, node_features, c1_W1, c1_b1, c1_g, c1_be, c1_W2, c1_b2, c2_W1, c2_b1, c2_g, c2_be, c2_W2, c2_b2, lin1_W, lin1_b, lin2_W, lin2_b, ep_W1, ep_b1, ep_W2, ep_b2):
    raise NotImplementedError("write your pallas kernel here")


# speedup vs baseline: 294.2317x; 294.2317x over previous
import math

import jax
import jax.numpy as jnp
from jax.experimental import pallas as pl
from jax.experimental.pallas import tpu as pltpu

_N = 2048
_F = 16
_H = 64
_CSB = 256
_T = 256
_BN_SCALE = 1.0 / math.sqrt(1.0 + 1e-5)


def _cumsum_rows(x, tri, nblk, blk):
    carry = jnp.zeros((1, x.shape[1]), jnp.float32)
    outs = []
    for b in range(nblk):
        xb = x[b * blk:(b + 1) * blk, :]
        inc = jnp.dot(tri, xb, preferred_element_type=jnp.float32)
        outs.append(inc + carry)
        carry = carry + inc[blk - 1:blk, :]
    return jnp.concatenate(outs, axis=0)


def _prelude_kernel(nf_ref,
                    c1w1_ref, c1b1_ref, c1g_ref, c1be_ref, c1w2_ref, c1b2_ref,
                    c2w1_ref, c2b1_ref, c2g_ref, c2be_ref, c2w2_ref, c2b2_ref,
                    l1w_ref, l1b_ref, l2w_ref, l2b_ref,
                    epw1a_ref, epw1b_ref, epb1_ref,
                    a_ref, b_ref):
    ri = jax.lax.broadcasted_iota(jnp.int32, (_CSB, _CSB), 0)
    ci = jax.lax.broadcasted_iota(jnp.int32, (_CSB, _CSB), 1)
    tri = (ri >= ci).astype(jnp.float32)
    nblk = _N // _CSB

    def gin_mlp(h, w1, b1, g, be, w2, b2):
        h = jnp.dot(h, w1, preferred_element_type=jnp.float32) + b1
        h = h * (g * _BN_SCALE) + be
        h = jnp.maximum(h, 0.0)
        h = jnp.dot(h, w2, preferred_element_type=jnp.float32) + b2
        return jnp.maximum(h, 0.0)

    h = nf_ref[...]
    h = _cumsum_rows(h, tri, nblk, _CSB)
    h = gin_mlp(h, c1w1_ref[...], c1b1_ref[...], c1g_ref[...], c1be_ref[...],
                c1w2_ref[...], c1b2_ref[...])
    h = _cumsum_rows(h, tri, nblk, _CSB)
    h = gin_mlp(h, c2w1_ref[...], c2b1_ref[...], c2g_ref[...], c2be_ref[...],
                c2w2_ref[...], c2b2_ref[...])
    h = jnp.dot(h, l1w_ref[...], preferred_element_type=jnp.float32) + l1b_ref[...]
    h = jnp.where(h >= 0.0, h, 0.01 * h)
    h = jnp.dot(h, l2w_ref[...], preferred_element_type=jnp.float32) + l2b_ref[...]
    a_ref[...] = jnp.dot(h, epw1a_ref[...], preferred_element_type=jnp.float32) + epb1_ref[...]
    b_ref[...] = jnp.dot(h, epw1b_ref[...], preferred_element_type=jnp.float32)


def _pair_kernel(ar_ref, br_ref, atc_ref, btc_ref, w2_ref, b2_ref, out_ref):
    i = pl.program_id(0)
    j = pl.program_id(1)
    b2 = b2_ref[0]

    def accum(col_src, row_src):
        acc = jnp.full((_T, _T), b2, jnp.float32)
        for h in range(_H):
            s = col_src[:, h:h + 1] + row_src[h:h + 1, :]
            acc = acc + jnp.maximum(s, 0.0) * w2_ref[h]
        return acc

    @pl.when(i < j)
    def _():
        out_ref[...] = jax.nn.sigmoid(accum(ar_ref[...], btc_ref[...]))

    @pl.when(i > j)
    def _():
        out_ref[...] = jax.nn.sigmoid(accum(br_ref[...], atc_ref[...]))

    @pl.when(i == j)
    def _():
        up = jax.nn.sigmoid(accum(ar_ref[...], btc_ref[...]))
        lo = jax.nn.sigmoid(accum(br_ref[...], atc_ref[...]))
        ri = jax.lax.broadcasted_iota(jnp.int32, (_T, _T), 0)
        ci = jax.lax.broadcasted_iota(jnp.int32, (_T, _T), 1)
        out_ref[...] = jnp.where(ri < ci, up, jnp.where(ri > ci, lo, 0.0))


def kernel(x, node_features,
           c1_W1, c1_b1, c1_g, c1_be, c1_W2, c1_b2,
           c2_W1, c2_b1, c2_g, c2_be, c2_W2, c2_b2,
           lin1_W, lin1_b, lin2_W, lin2_b,
           ep_W1, ep_b1, ep_W2, ep_b2):
    del x
    r = lambda v: v.reshape(1, -1)
    a, b = pl.pallas_call(
        _prelude_kernel,
        out_shape=(
            jax.ShapeDtypeStruct((_N, _H), jnp.float32),
            jax.ShapeDtypeStruct((_N, _H), jnp.float32),
        ),
    )(node_features,
      c1_W1, r(c1_b1), r(c1_g), r(c1_be), c1_W2, r(c1_b2),
      c2_W1, r(c2_b1), r(c2_g), r(c2_be), c2_W2, r(c2_b2),
      lin1_W, r(lin1_b), lin2_W, r(lin2_b),
      ep_W1[:_F], ep_W1[_F:], r(ep_b1))

    at = a.T
    bt = b.T
    nt = _N // _T
    probs = pl.pallas_call(
        _pair_kernel,
        grid=(nt, nt),
        in_specs=[
            pl.BlockSpec((_T, _H), lambda i, j: (i, 0)),
            pl.BlockSpec((_T, _H), lambda i, j: (i, 0)),
            pl.BlockSpec((_H, _T), lambda i, j: (0, j)),
            pl.BlockSpec((_H, _T), lambda i, j: (0, j)),
            pl.BlockSpec(memory_space=pltpu.SMEM),
            pl.BlockSpec(memory_space=pltpu.SMEM),
        ],
        out_specs=pl.BlockSpec((_T, _T), lambda i, j: (i, j)),
        out_shape=jax.ShapeDtypeStruct((_N, _N), jnp.float32),
    )(a, b, at, bt, ep_W2[:, 0], ep_b2)
    return probs


# speedup vs baseline: 429.2596x; 1.4589x over previous
import math

import numpy as np
import jax
import jax.numpy as jnp
from jax.experimental import pallas as pl
from jax.experimental.pallas import tpu as pltpu

_N = 2048
_F = 16
_H = 64
_CSB = 256
_T = 256
_BN_SCALE = 1.0 / math.sqrt(1.0 + 1e-5)


def _cumsum_rows(x, tri, nblk, blk):
    carry = jnp.zeros((1, x.shape[1]), jnp.float32)
    outs = []
    for b in range(nblk):
        xb = x[b * blk:(b + 1) * blk, :]
        inc = jnp.dot(tri, xb, preferred_element_type=jnp.float32)
        outs.append(inc + carry)
        carry = carry + inc[blk - 1:blk, :]
    return jnp.concatenate(outs, axis=0)


def _prelude_kernel(nf_ref,
                    c1w1_ref, c1b1_ref, c1g_ref, c1be_ref, c1w2_ref, c1b2_ref,
                    c2w1_ref, c2b1_ref, c2g_ref, c2be_ref, c2w2_ref, c2b2_ref,
                    l1w_ref, l1b_ref, l2w_ref, l2b_ref,
                    epw1a_ref, epw1b_ref, epb1_ref,
                    a_ref, b_ref):
    ri = jax.lax.broadcasted_iota(jnp.int32, (_CSB, _CSB), 0)
    ci = jax.lax.broadcasted_iota(jnp.int32, (_CSB, _CSB), 1)
    tri = (ri >= ci).astype(jnp.float32)
    nblk = _N // _CSB

    def gin_mlp(h, w1, b1, g, be, w2, b2):
        h = jnp.dot(h, w1, preferred_element_type=jnp.float32) + b1
        h = h * (g * _BN_SCALE) + be
        h = jnp.maximum(h, 0.0)
        h = jnp.dot(h, w2, preferred_element_type=jnp.float32) + b2
        return jnp.maximum(h, 0.0)

    h = nf_ref[...]
    h = _cumsum_rows(h, tri, nblk, _CSB)
    h = gin_mlp(h, c1w1_ref[...], c1b1_ref[...], c1g_ref[...], c1be_ref[...],
                c1w2_ref[...], c1b2_ref[...])
    h = _cumsum_rows(h, tri, nblk, _CSB)
    h = gin_mlp(h, c2w1_ref[...], c2b1_ref[...], c2g_ref[...], c2be_ref[...],
                c2w2_ref[...], c2b2_ref[...])
    h = jnp.dot(h, l1w_ref[...], preferred_element_type=jnp.float32) + l1b_ref[...]
    h = jnp.where(h >= 0.0, h, 0.01 * h)
    h = jnp.dot(h, l2w_ref[...], preferred_element_type=jnp.float32) + l2b_ref[...]
    a_ref[...] = jnp.dot(h, epw1a_ref[...], preferred_element_type=jnp.float32) + epb1_ref[...]
    b_ref[...] = jnp.dot(h, epw1b_ref[...], preferred_element_type=jnp.float32)


def _pair_kernel(bi_ref, bj_ref, w2_ref, b2_ref, ar_ref, btc_ref, out_ref,
                 tile_ref, mirror_ref, sem0, sem1):
    t = pl.program_id(0)
    bi = bi_ref[t]
    bj = bj_ref[t]

    acc = jnp.full((_T, _T), b2_ref[0], jnp.float32)
    col_src = ar_ref[...]
    row_src = btc_ref[...]
    for h in range(_H):
        s = col_src[:, h:h + 1] + row_src[h:h + 1, :]
        acc = acc + jnp.maximum(s, 0.0) * w2_ref[h]
    up = jax.nn.sigmoid(acc)

    @pl.when(bi == bj)
    def _():
        ri = jax.lax.broadcasted_iota(jnp.int32, (_T, _T), 0)
        ci = jax.lax.broadcasted_iota(jnp.int32, (_T, _T), 1)
        m = jnp.where(ri < ci, up, 0.0)
        tile_ref[...] = m + m.T
        cp = pltpu.make_async_copy(
            tile_ref, out_ref.at[pl.ds(bi * _T, _T), pl.ds(bj * _T, _T)], sem0)
        cp.start()
        cp.wait()

    @pl.when(bi != bj)
    def _():
        tile_ref[...] = up
        mirror_ref[...] = up.T
        cp0 = pltpu.make_async_copy(
            tile_ref, out_ref.at[pl.ds(bi * _T, _T), pl.ds(bj * _T, _T)], sem0)
        cp1 = pltpu.make_async_copy(
            mirror_ref, out_ref.at[pl.ds(bj * _T, _T), pl.ds(bi * _T, _T)], sem1)
        cp0.start()
        cp1.start()
        cp0.wait()
        cp1.wait()


def kernel(x, node_features,
           c1_W1, c1_b1, c1_g, c1_be, c1_W2, c1_b2,
           c2_W1, c2_b1, c2_g, c2_be, c2_W2, c2_b2,
           lin1_W, lin1_b, lin2_W, lin2_b,
           ep_W1, ep_b1, ep_W2, ep_b2):
    del x
    r = lambda v: v.reshape(1, -1)
    a, b = pl.pallas_call(
        _prelude_kernel,
        out_shape=(
            jax.ShapeDtypeStruct((_N, _H), jnp.float32),
            jax.ShapeDtypeStruct((_N, _H), jnp.float32),
        ),
    )(node_features,
      c1_W1, r(c1_b1), r(c1_g), r(c1_be), c1_W2, r(c1_b2),
      c2_W1, r(c2_b1), r(c2_g), r(c2_be), c2_W2, r(c2_b2),
      lin1_W, r(lin1_b), lin2_W, r(lin2_b),
      ep_W1[:_F], ep_W1[_F:], r(ep_b1))

    bt = b.T
    nt = _N // _T
    bi_tbl, bj_tbl = np.triu_indices(nt, k=0)
    probs = pl.pallas_call(
        _pair_kernel,
        grid_spec=pltpu.PrefetchScalarGridSpec(
            num_scalar_prefetch=4,
            grid=(len(bi_tbl),),
            in_specs=[
                pl.BlockSpec((_T, _H), lambda t, bi, bj, w2, b2: (bi[t], 0)),
                pl.BlockSpec((_H, _T), lambda t, bi, bj, w2, b2: (0, bj[t])),
            ],
            out_specs=pl.BlockSpec(memory_space=pl.ANY),
            scratch_shapes=[
                pltpu.VMEM((_T, _T), jnp.float32),
                pltpu.VMEM((_T, _T), jnp.float32),
                pltpu.SemaphoreType.DMA,
                pltpu.SemaphoreType.DMA,
            ],
        ),
        out_shape=jax.ShapeDtypeStruct((_N, _N), jnp.float32),
    )(jnp.asarray(bi_tbl, jnp.int32), jnp.asarray(bj_tbl, jnp.int32),
      ep_W2[:, 0], ep_b2, a, bt)
    return probs


# speedup vs baseline: 524.6828x; 1.2223x over previous
import math

import numpy as np
import jax
import jax.numpy as jnp
from jax.experimental import pallas as pl
from jax.experimental.pallas import tpu as pltpu

_N = 2048
_F = 16
_H = 64
_CSB = 256
_T = 256
_BN_SCALE = 1.0 / math.sqrt(1.0 + 1e-5)


def _cumsum_rows(x, tri, nblk, blk):
    carry = jnp.zeros((1, x.shape[1]), jnp.float32)
    outs = []
    for b in range(nblk):
        xb = x[b * blk:(b + 1) * blk, :]
        inc = jnp.dot(tri, xb, preferred_element_type=jnp.float32)
        outs.append(inc + carry)
        carry = carry + inc[blk - 1:blk, :]
    return jnp.concatenate(outs, axis=0)


def _prelude_kernel(nf_ref,
                    c1w1_ref, c1b1_ref, c1g_ref, c1be_ref, c1w2_ref, c1b2_ref,
                    c2w1_ref, c2b1_ref, c2g_ref, c2be_ref, c2w2_ref, c2b2_ref,
                    l1w_ref, l1b_ref, l2w_ref, l2b_ref,
                    epw1a_ref, epw1b_ref, epb1_ref,
                    a_ref, b_ref):
    ri = jax.lax.broadcasted_iota(jnp.int32, (_CSB, _CSB), 0)
    ci = jax.lax.broadcasted_iota(jnp.int32, (_CSB, _CSB), 1)
    tri = (ri >= ci).astype(jnp.float32)
    nblk = _N // _CSB

    def gin_mlp(h, w1, b1, g, be, w2, b2):
        h = jnp.dot(h, w1, preferred_element_type=jnp.float32) + b1
        h = h * (g * _BN_SCALE) + be
        h = jnp.maximum(h, 0.0)
        h = jnp.dot(h, w2, preferred_element_type=jnp.float32) + b2
        return jnp.maximum(h, 0.0)

    h = nf_ref[...]
    h = _cumsum_rows(h, tri, nblk, _CSB)
    h = gin_mlp(h, c1w1_ref[...], c1b1_ref[...], c1g_ref[...], c1be_ref[...],
                c1w2_ref[...], c1b2_ref[...])
    h = _cumsum_rows(h, tri, nblk, _CSB)
    h = gin_mlp(h, c2w1_ref[...], c2b1_ref[...], c2g_ref[...], c2be_ref[...],
                c2w2_ref[...], c2b2_ref[...])
    h = jnp.dot(h, l1w_ref[...], preferred_element_type=jnp.float32) + l1b_ref[...]
    h = jnp.where(h >= 0.0, h, 0.01 * h)
    h = jnp.dot(h, l2w_ref[...], preferred_element_type=jnp.float32) + l2b_ref[...]
    a_ref[...] = jnp.dot(h, epw1a_ref[...], preferred_element_type=jnp.float32) + epb1_ref[...]
    b_ref[...] = jnp.dot(h, epw1b_ref[...], preferred_element_type=jnp.float32)


_NSTEPS = (_N // _T) * (_N // _T + 1) // 2


def _pair_kernel(bi_ref, bj_ref, w2_ref, b2_ref, ar_ref, btc_ref, out_ref,
                 tile_ref, mirror_ref, sems):
    t = pl.program_id(0)
    bi = bi_ref[t]
    bj = bj_ref[t]
    slot = jax.lax.rem(t, 2)

    def copies(s, pi, pj):
        c0 = pltpu.make_async_copy(
            tile_ref.at[s],
            out_ref.at[pl.ds(pi * _T, _T), pl.ds(pj * _T, _T)],
            sems.at[s, 0])
        c1 = pltpu.make_async_copy(
            mirror_ref.at[s],
            out_ref.at[pl.ds(pj * _T, _T), pl.ds(pi * _T, _T)],
            sems.at[s, 1])
        return c0, c1

    acc = jnp.full((_T, _T), b2_ref[0], jnp.float32)
    col_src = ar_ref[...]
    row_src = btc_ref[...]
    for h in range(_H):
        s = col_src[:, h:h + 1] + row_src[h:h + 1, :]
        acc = acc + jnp.maximum(s, 0.0) * w2_ref[h]
    up = jax.nn.sigmoid(acc)

    @pl.when(t >= 2)
    def _():
        c0, c1 = copies(slot, bi_ref[t - 2], bj_ref[t - 2])
        c0.wait()
        c1.wait()

    @pl.when(bi == bj)
    def _():
        ri = jax.lax.broadcasted_iota(jnp.int32, (_T, _T), 0)
        ci = jax.lax.broadcasted_iota(jnp.int32, (_T, _T), 1)
        m = jnp.where(ri < ci, up, 0.0)
        sym = m + m.T
        tile_ref[slot] = sym
        mirror_ref[slot] = sym
    @pl.when(bi != bj)
    def _():
        tile_ref[slot] = up
        mirror_ref[slot] = up.T

    c0, c1 = copies(slot, bi, bj)
    c0.start()
    c1.start()

    @pl.when(t == _NSTEPS - 1)
    def _():
        c0.wait()
        c1.wait()
        p0, p1 = copies(1 - slot, bi_ref[t - 1], bj_ref[t - 1])
        p0.wait()
        p1.wait()


def kernel(x, node_features,
           c1_W1, c1_b1, c1_g, c1_be, c1_W2, c1_b2,
           c2_W1, c2_b1, c2_g, c2_be, c2_W2, c2_b2,
           lin1_W, lin1_b, lin2_W, lin2_b,
           ep_W1, ep_b1, ep_W2, ep_b2):
    del x
    r = lambda v: v.reshape(1, -1)
    a, b = pl.pallas_call(
        _prelude_kernel,
        out_shape=(
            jax.ShapeDtypeStruct((_N, _H), jnp.float32),
            jax.ShapeDtypeStruct((_N, _H), jnp.float32),
        ),
    )(node_features,
      c1_W1, r(c1_b1), r(c1_g), r(c1_be), c1_W2, r(c1_b2),
      c2_W1, r(c2_b1), r(c2_g), r(c2_be), c2_W2, r(c2_b2),
      lin1_W, r(lin1_b), lin2_W, r(lin2_b),
      ep_W1[:_F], ep_W1[_F:], r(ep_b1))

    bt = b.T
    nt = _N // _T
    bi_tbl, bj_tbl = np.triu_indices(nt, k=0)
    probs = pl.pallas_call(
        _pair_kernel,
        grid_spec=pltpu.PrefetchScalarGridSpec(
            num_scalar_prefetch=4,
            grid=(len(bi_tbl),),
            in_specs=[
                pl.BlockSpec((_T, _H), lambda t, bi, bj, w2, b2: (bi[t], 0)),
                pl.BlockSpec((_H, _T), lambda t, bi, bj, w2, b2: (0, bj[t])),
            ],
            out_specs=pl.BlockSpec(memory_space=pl.ANY),
            scratch_shapes=[
                pltpu.VMEM((2, _T, _T), jnp.float32),
                pltpu.VMEM((2, _T, _T), jnp.float32),
                pltpu.SemaphoreType.DMA((2, 2)),
            ],
        ),
        out_shape=jax.ShapeDtypeStruct((_N, _N), jnp.float32),
    )(jnp.asarray(bi_tbl, jnp.int32), jnp.asarray(bj_tbl, jnp.int32),
      ep_W2[:, 0], ep_b2, a, bt)
    return probs
